# Initial kernel scaffold; baseline (speedup 1.0000x reference)
import jax
import jax.numpy as jnp
from jax.experimental import pallas as pl


def kernel(features, src_nodes, s2s1, s2d1, dif1, s2s2, s2d2, dif2, W1, W2, attention_vec, relation_vectors, Wc):
    raise NotImplementedError("write your pallas kernel here")


# speedup vs baseline: 3.4704x; 3.4704x over previous
import functools

import jax
import jax.numpy as jnp
from jax import lax
from jax.experimental import pallas as pl
from jax.experimental.pallas import tpu as pltpu
from jax.experimental.pallas import tpu_sc as plsc

F = 256
NREL = 3
NSRC = 8192
NDST = 1024
N2 = 1024
N3 = 256
H = 128
BK = 2048
KSTEPS = NSRC // BK

NW = 32
SRC_PER_W = NSRC // NW
DST_PER_W = NDST // NW
GCHUNK = 128


def _sc_gather(features, srcn_flat, s2s1_flat, s2d1_flat):
    mesh = plsc.VectorSubcoreMesh(core_axis_name="c", subcore_axis_name="s")
    NCH = NREL * (SRC_PER_W // GCHUNK)

    @functools.partial(
        pl.kernel,
        mesh=mesh,
        out_type=(
            jax.ShapeDtypeStruct((NREL * NSRC, F), jnp.float32),
            jax.ShapeDtypeStruct((NREL * NDST, F), jnp.float32),
        ),
        scratch_types=[
            pltpu.VMEM((NREL * NSRC,), jnp.int32),
            pltpu.VMEM((NREL * SRC_PER_W,), jnp.int32),
            pltpu.VMEM((NREL * DST_PER_W,), jnp.int32),
        ] + [pltpu.VMEM((GCHUNK,), jnp.int32) for _ in range(NCH)]
          + [pltpu.VMEM((DST_PER_W,), jnp.int32) for _ in range(NREL)]
          + [
            pltpu.VMEM((GCHUNK, F), jnp.float32),
            pltpu.VMEM((GCHUNK, F), jnp.float32),
            pltpu.VMEM((DST_PER_W, F), jnp.float32),
            pltpu.VMEM((DST_PER_W, F), jnp.float32),
            pltpu.SemaphoreType.DMA,
            pltpu.SemaphoreType.DMA,
            pltpu.SemaphoreType.DMA,
            pltpu.SemaphoreType.DMA,
            pltpu.SemaphoreType.DMA,
        ],
        compiler_params=pltpu.CompilerParams(needs_layout_passes=False),
    )
    def k(feat_hbm, srcn_hbm, s2s1_hbm, s2d1_hbm, osrc_hbm, odst_hbm, *refs):
        srcn_v, idxs_v, idxd_v = refs[0], refs[1], refs[2]
        cbufs = refs[3:3 + NCH]
        dbufs = refs[3 + NCH:3 + NCH + NREL]
        rows0, rows1, rowsd0, rowsd1 = refs[3 + NCH + NREL:7 + NCH + NREL]
        sem_st, sg0, sg1, so0, so1 = refs[7 + NCH + NREL:]
        rbufs, sgs, sos = (rows0, rows1), (sg0, sg1), (so0, so1)
        dbufs_rows, dsems = (rowsd0, rowsd1), (sg0, sg1)
        wid = lax.axis_index("s") * 2 + lax.axis_index("c")

        stage = [pltpu.async_copy(srcn_hbm, srcn_v, sem_st)]
        for i in range(NREL):
            stage.append(pltpu.async_copy(
                s2s1_hbm.at[pl.ds(i * NSRC + wid * SRC_PER_W, SRC_PER_W)],
                idxs_v.at[pl.ds(i * SRC_PER_W, SRC_PER_W)], sem_st))
            stage.append(pltpu.async_copy(
                s2d1_hbm.at[pl.ds(i * NDST + wid * DST_PER_W, DST_PER_W)],
                idxd_v.at[pl.ds(i * DST_PER_W, DST_PER_W)], sem_st))
        for h in stage:
            h.wait()

        for i in range(NREL):
            for c in range(SRC_PER_W // GCHUNK):
                buf = cbufs[i * (SRC_PER_W // GCHUNK) + c]
                for j in range(GCHUNK // 16):
                    ids = idxs_v[pl.ds(i * SRC_PER_W + c * GCHUNK + j * 16, 16)]
                    buf[pl.ds(j * 16, 16)] = plsc.load_gather(srcn_v, [ids + i * NSRC])
            for j in range(DST_PER_W // 16):
                ids = idxd_v[pl.ds(i * DST_PER_W + j * 16, 16)]
                dbufs[i][pl.ds(j * 16, 16)] = plsc.load_gather(srcn_v, [ids + i * NSRC])

        def osl(ci):
            i, c = divmod(ci, SRC_PER_W // GCHUNK)
            return osrc_hbm.at[pl.ds(i * NSRC + wid * SRC_PER_W + c * GCHUNK, GCHUNK)]

        hg = [None] * NCH
        ho = [None] * NCH
        hg[0] = pltpu.async_copy(feat_hbm.at[cbufs[0]], rows0, sg0)
        hg[1] = pltpu.async_copy(feat_hbm.at[cbufs[1]], rows1, sg1)
        for ci in range(NCH):
            p = ci % 2
            hg[ci].wait()
            ho[ci] = pltpu.async_copy(rbufs[p], osl(ci), sos[p])
            if ci + 2 < NCH:
                ho[ci].wait()
                hg[ci + 2] = pltpu.async_copy(feat_hbm.at[cbufs[ci + 2]], rbufs[p], sgs[p])
        ho[NCH - 2].wait()
        ho[NCH - 1].wait()

        hd = [None] * NREL
        od = [None] * NREL
        hd[0] = pltpu.async_copy(feat_hbm.at[dbufs[0]], rowsd0, sg0)
        hd[1] = pltpu.async_copy(feat_hbm.at[dbufs[1]], rowsd1, sg1)
        for i in range(NREL):
            p = i % 2
            hd[i].wait()
            od[i] = pltpu.async_copy(
                dbufs_rows[p], odst_hbm.at[pl.ds(i * NDST + wid * DST_PER_W, DST_PER_W)],
                sos[p])
            if i + 2 < NREL:
                od[i].wait()
                hd[i + 2] = pltpu.async_copy(feat_hbm.at[dbufs[i + 2]], dbufs_rows[p], dsems[p])
        od[NREL - 2].wait()
        od[NREL - 1].wait()

    return k(features, srcn_flat, s2s1_flat, s2d1_flat)


def _dot(a, b):
    return jax.lax.dot_general(a, b, (((1,), (0,)), ((), ())),
                               preferred_element_type=jnp.float32)


def _tc_body(dif1_ref, srcg_ref, dstg_ref, dif2_ref, s2s2_ref, s2d2_ref,
             w1a_ref, w1b_ref, w2a_ref, w2b_ref, a1_ref, wc_ref,
             out_ref, acc_ref, xsum_ref):
    i = pl.program_id(0)
    k = pl.program_id(1)

    part = _dot(dif1_ref[0], srcg_ref[0])

    @pl.when(k == 0)
    def _():
        acc_ref[...] = part

    @pl.when(k != 0)
    def _():
        acc_ref[...] += part

    @pl.when(k == KSTEPS - 1)
    def _():
        agg1 = acc_ref[...]
        h1 = _dot(agg1, w1a_ref[...]) + _dot(dstg_ref[0], w1b_ref[...])
        logits1 = _dot(h1, a1_ref[...])
        e1 = jnp.exp(logits1 - jnp.max(logits1, axis=0, keepdims=True))
        att1 = e1 / jnp.sum(e1, axis=0, keepdims=True)
        x1 = h1 * att1

        cols2 = lax.broadcasted_iota(jnp.int32, (N2, N2), 1)
        e2 = (cols2 == s2s2_ref[0]).astype(jnp.float32)
        src2 = _dot(e2, x1)
        agg2 = _dot(dif2_ref[0], src2)
        colsd = lax.broadcasted_iota(jnp.int32, (N3, N2), 1)
        ed = (colsd == s2d2_ref[0]).astype(jnp.float32)
        dst2 = _dot(ed, x1)
        h2 = _dot(agg2, w2a_ref[...]) + _dot(dst2, w2b_ref[...])
        logits2 = _dot(h2, a1_ref[...])
        g2 = jnp.exp(logits2 - jnp.max(logits2, axis=0, keepdims=True))
        att2 = g2 / jnp.sum(g2, axis=0, keepdims=True)
        x2 = h2 * att2

        @pl.when(i == 0)
        def _():
            xsum_ref[...] = x2

        @pl.when(i != 0)
        def _():
            xsum_ref[...] += x2

        @pl.when(i == NREL - 1)
        def _():
            s = xsum_ref[...]
            ss = jnp.sum(s * s, axis=1, keepdims=True)
            nrm = s * lax.rsqrt(jnp.maximum(ss, 1e-12))
            lc = _dot(nrm, wc_ref[...])
            ec = jnp.exp(lc - jnp.max(lc, axis=1, keepdims=True))
            out_ref[...] = ec / jnp.sum(ec, axis=1, keepdims=True)


def _tc_forward(srcg, dstg, dif1, dif2, s2s2r, s2d2r, W1a, W1b, W2a, W2b, a1, Wc):
    return pl.pallas_call(
        _tc_body,
        grid=(NREL, KSTEPS),
        in_specs=[
            pl.BlockSpec((1, NDST, BK), lambda i, k: (i, 0, k)),
            pl.BlockSpec((1, BK, F), lambda i, k: (i, k, 0)),
            pl.BlockSpec((1, NDST, F), lambda i, k: (i, 0, 0)),
            pl.BlockSpec((1, N3, N2), lambda i, k: (i, 0, 0)),
            pl.BlockSpec((1, N2, 1), lambda i, k: (i, 0, 0)),
            pl.BlockSpec((1, N3, 1), lambda i, k: (i, 0, 0)),
            pl.BlockSpec((F, H), lambda i, k: (0, 0)),
            pl.BlockSpec((F, H), lambda i, k: (0, 0)),
            pl.BlockSpec((H, H), lambda i, k: (0, 0)),
            pl.BlockSpec((H, H), lambda i, k: (0, 0)),
            pl.BlockSpec((H, 1), lambda i, k: (0, 0)),
            pl.BlockSpec((H, 2), lambda i, k: (0, 0)),
        ],
        out_specs=pl.BlockSpec((N3, 2), lambda i, k: (0, 0)),
        out_shape=jax.ShapeDtypeStruct((N3, 2), jnp.float32),
        scratch_shapes=[
            pltpu.VMEM((NDST, F), jnp.float32),
            pltpu.VMEM((N3, H), jnp.float32),
        ],
        compiler_params=pltpu.CompilerParams(
            dimension_semantics=("arbitrary", "arbitrary"),
        ),
    )(dif1, srcg, dstg, dif2, s2s2r, s2d2r, W1a, W1b, W2a, W2b, a1, Wc)


def kernel(features, src_nodes, s2s1, s2d1, dif1, s2s2, s2d2, dif2,
           W1, W2, attention_vec, relation_vectors, Wc):
    del relation_vectors
    srcn_flat = src_nodes.reshape(-1).astype(jnp.int32)
    s2s1_flat = s2s1.reshape(-1).astype(jnp.int32)
    s2d1_flat = s2d1.reshape(-1).astype(jnp.int32)
    osrc, odst = _sc_gather(features, srcn_flat, s2s1_flat, s2d1_flat)
    srcg = osrc.reshape(NREL, NSRC, F)
    dstg = odst.reshape(NREL, NDST, F)
    s2s2r = s2s2.astype(jnp.int32).reshape(NREL, N2, 1)
    s2d2r = s2d2.astype(jnp.int32).reshape(NREL, N3, 1)
    W1a, W1b = W1[:F], W1[F:]
    W2a, W2b = W2[:H], W2[H:]
    a1 = attention_vec[:H]
    return _tc_forward(srcg, dstg, dif1, dif2, s2s2r, s2d2r,
                       W1a, W1b, W2a, W2b, a1, Wc)
